# baseline (speedup 1.0000x reference)
import jax
import jax.numpy as jnp
import numpy as np
from jax.experimental import pallas as pl
from jax.experimental.pallas import tpu as pltpu

_ANCHORS_ALL = np.array(
    [[10, 13], [16, 30], [33, 23], [30, 61], [62, 45], [59, 119],
     [116, 90], [156, 198], [373, 326]], dtype=np.float32)
_ANCHORS_MASK = np.array([0, 1, 2], dtype=np.int32)
_NUM_CLASSES = 80
_C = 5 + _NUM_CLASSES
_NA = 3

_NB, _NCH, _NH, _NW = 16, 255, 64, 64
_HBLK = 8

_AW = _ANCHORS_ALL[_ANCHORS_MASK, 0]
_AH = _ANCHORS_ALL[_ANCHORS_MASK, 1]


def _decode_kernel(stride_ref, x_ref, o_ref):
    j = pl.program_id(1)
    v = x_ref[0]
    t3 = jnp.transpose(v, (1, 2, 0)).reshape(_HBLK * _NW, _NA * _C)
    st = jnp.stack([t3[:, _C * a:_C * (a + 1)] for a in range(_NA)], axis=1)
    t = st.reshape(_HBLK * _NW * _NA, _C)

    rows = _HBLK * _NW * _NA
    n = jax.lax.broadcasted_iota(jnp.int32, (rows, 1), 0)
    c = jax.lax.broadcasted_iota(jnp.int32, (1, _C), 1)

    a = n % _NA
    w_f = ((n // _NA) % _NW).astype(jnp.float32)
    h_f = (n // (_NA * _NW) + j * _HBLK).astype(jnp.float32)

    is_wh = jnp.logical_or(c == 2, c == 3)
    e = jnp.exp(jnp.where(is_wh, t, -t))
    sig = 1.0 / (1.0 + e)

    mesh = jnp.where(c == 0, w_f, h_f)
    mul_w = jnp.where(a == 0, _AW[0], jnp.where(a == 1, _AW[1], _AW[2]))
    mul_h = jnp.where(a == 0, _AH[0], jnp.where(a == 1, _AH[1], _AH[2]))
    mul = jnp.where(c == 2, mul_w, mul_h)

    stride = stride_ref[0, 0]
    res = jnp.where(c < 2, (sig + mesh) * stride,
                    jnp.where(c < 4, e * mul, sig))
    o_ref[0] = res


def kernel(x, img_dim):
    nB, nCh, nH, nW = x.shape
    stride = (img_dim[1].astype(jnp.float32) / nH).reshape(1, 1)
    grid = (nB, nH // _HBLK)
    out = pl.pallas_call(
        _decode_kernel,
        grid=grid,
        in_specs=[
            pl.BlockSpec((1, 1), lambda b, j: (0, 0),
                         memory_space=pltpu.SMEM),
            pl.BlockSpec((1, nCh, _HBLK, nW), lambda b, j: (b, 0, j, 0)),
        ],
        out_specs=pl.BlockSpec((1, _HBLK * nW * _NA, _C),
                               lambda b, j: (b, j, 0)),
        out_shape=jax.ShapeDtypeStruct((nB, nH * nW * _NA, _C), x.dtype),
    )(stride, x)
    return out


# speedup vs baseline: 3.2786x; 3.2786x over previous
import jax
import jax.numpy as jnp
import numpy as np
from jax.experimental import pallas as pl
from jax.experimental.pallas import tpu as pltpu

_ANCHORS_ALL = np.array(
    [[10, 13], [16, 30], [33, 23], [30, 61], [62, 45], [59, 119],
     [116, 90], [156, 198], [373, 326]], dtype=np.float32)
_ANCHORS_MASK = np.array([0, 1, 2], dtype=np.int32)
_NUM_CLASSES = 80
_C = 5 + _NUM_CLASSES
_NA = 3

_NB, _NCH, _NH, _NW = 16, 255, 64, 64
_HBLK = 8

_AW = _ANCHORS_ALL[_ANCHORS_MASK, 0]
_AH = _ANCHORS_ALL[_ANCHORS_MASK, 1]


def _decode_kernel(stride_ref, x_ref, o_ref):
    j = pl.program_id(1)
    v = x_ref[0]
    hw = _HBLK * _NW
    v2d = v.reshape(_NCH, hw)
    eye_c = jnp.eye(_C, dtype=jnp.float32)
    dn = (((0,), (0,)), ((), ()))

    n = jax.lax.broadcasted_iota(jnp.int32, (hw, 1), 0)
    c = jax.lax.broadcasted_iota(jnp.int32, (1, _C), 1)
    w_f = (n % _NW).astype(jnp.float32)
    h_f = (n // _NW + j * _HBLK).astype(jnp.float32)
    mesh = jnp.where(c == 0, w_f, h_f)
    is_wh = jnp.logical_or(c == 2, c == 3)
    stride = stride_ref[0, 0]

    for a in range(_NA):
        slab = jax.lax.dot_general(v2d[_C * a:_C * (a + 1), :], eye_c, dn,
                                   precision=jax.lax.Precision.HIGHEST,
                                   preferred_element_type=jnp.float32)
        e = jnp.exp(jnp.where(is_wh, slab, -slab))
        sig = 1.0 / (1.0 + e)
        mul = jnp.where(c == 2, _AW[a], _AH[a])
        res = jnp.where(c < 2, (sig + mesh) * stride,
                        jnp.where(c < 4, e * mul, sig))
        o_ref[0, pl.Slice(a, hw, _NA), :] = res


def kernel(x, img_dim):
    nB, nCh, nH, nW = x.shape
    stride = (img_dim[1].astype(jnp.float32) / nH).reshape(1, 1)
    grid = (nB, nH // _HBLK)
    out = pl.pallas_call(
        _decode_kernel,
        grid=grid,
        in_specs=[
            pl.BlockSpec((1, 1), lambda b, j: (0, 0),
                         memory_space=pltpu.SMEM),
            pl.BlockSpec((1, nCh, _HBLK, nW), lambda b, j: (b, 0, j, 0)),
        ],
        out_specs=pl.BlockSpec((1, _HBLK * nW * _NA, _C),
                               lambda b, j: (b, j, 0)),
        out_shape=jax.ShapeDtypeStruct((nB, nH * nW * _NA, _C), x.dtype),
    )(stride, x)
    return out


# speedup vs baseline: 3.5034x; 1.0685x over previous
import jax
import jax.numpy as jnp
import numpy as np
from jax.experimental import pallas as pl
from jax.experimental.pallas import tpu as pltpu

_ANCHORS_ALL = np.array(
    [[10, 13], [16, 30], [33, 23], [30, 61], [62, 45], [59, 119],
     [116, 90], [156, 198], [373, 326]], dtype=np.float32)
_ANCHORS_MASK = np.array([0, 1, 2], dtype=np.int32)
_NUM_CLASSES = 80
_C = 5 + _NUM_CLASSES
_NA = 3

_NB, _NCH, _NH, _NW = 16, 255, 64, 64
_HBLK = 64

_AW = _ANCHORS_ALL[_ANCHORS_MASK, 0]
_AH = _ANCHORS_ALL[_ANCHORS_MASK, 1]


def _decode_kernel(stride_ref, x_ref, o_ref):
    j = pl.program_id(1)
    v = x_ref[0]
    hw = _HBLK * _NW
    v2d = v.reshape(_NCH, hw)
    eye_c = jnp.eye(_C, dtype=jnp.float32)
    dn = (((0,), (0,)), ((), ()))

    n = jax.lax.broadcasted_iota(jnp.int32, (hw, 1), 0)
    c = jax.lax.broadcasted_iota(jnp.int32, (1, _C), 1)
    w_f = (n % _NW).astype(jnp.float32)
    h_f = (n // _NW + j * _HBLK).astype(jnp.float32)
    mesh = jnp.where(c == 0, w_f, h_f)
    is_wh = jnp.logical_or(c == 2, c == 3)
    stride = stride_ref[0, 0]

    for a in range(_NA):
        slab = jax.lax.dot_general(v2d[_C * a:_C * (a + 1), :], eye_c, dn,
                                   precision=jax.lax.Precision.HIGHEST,
                                   preferred_element_type=jnp.float32)
        e = jnp.exp(jnp.where(is_wh, slab, -slab))
        sig = 1.0 / (1.0 + e)
        mul = jnp.where(c == 2, _AW[a], _AH[a])
        res = jnp.where(c < 2, (sig + mesh) * stride,
                        jnp.where(c < 4, e * mul, sig))
        o_ref[0, pl.Slice(a, hw, _NA), :] = res


def kernel(x, img_dim):
    nB, nCh, nH, nW = x.shape
    stride = (img_dim[1].astype(jnp.float32) / nH).reshape(1, 1)
    grid = (nB, nH // _HBLK)
    out = pl.pallas_call(
        _decode_kernel,
        grid=grid,
        in_specs=[
            pl.BlockSpec((1, 1), lambda b, j: (0, 0),
                         memory_space=pltpu.SMEM),
            pl.BlockSpec((1, nCh, _HBLK, nW), lambda b, j: (b, 0, j, 0)),
        ],
        out_specs=pl.BlockSpec((1, _HBLK * nW * _NA, _C),
                               lambda b, j: (b, j, 0)),
        out_shape=jax.ShapeDtypeStruct((nB, nH * nW * _NA, _C), x.dtype),
    )(stride, x)
    return out


# speedup vs baseline: 3.5653x; 1.0177x over previous
import jax
import jax.numpy as jnp
import numpy as np
from jax.experimental import pallas as pl
from jax.experimental.pallas import tpu as pltpu

_ANCHORS_ALL = np.array(
    [[10, 13], [16, 30], [33, 23], [30, 61], [62, 45], [59, 119],
     [116, 90], [156, 198], [373, 326]], dtype=np.float32)
_ANCHORS_MASK = np.array([0, 1, 2], dtype=np.int32)
_NUM_CLASSES = 80
_C = 5 + _NUM_CLASSES
_NA = 3

_NB, _NCH, _NH, _NW = 16, 255, 64, 64
_HBLK = 16

_AW = _ANCHORS_ALL[_ANCHORS_MASK, 0]
_AH = _ANCHORS_ALL[_ANCHORS_MASK, 1]


def _decode_kernel(stride_ref, x_ref, o_ref):
    j = pl.program_id(1)
    v = x_ref[0]
    hw = _HBLK * _NW
    v2d = v.reshape(_NCH, hw)
    eye_c = jnp.eye(_C, dtype=jnp.float32)
    dn = (((0,), (0,)), ((), ()))

    n = jax.lax.broadcasted_iota(jnp.int32, (hw, 1), 0)
    c = jax.lax.broadcasted_iota(jnp.int32, (1, _C), 1)
    w_f = (n % _NW).astype(jnp.float32)
    h_f = (n // _NW + j * _HBLK).astype(jnp.float32)
    mesh = jnp.where(c == 0, w_f, h_f)
    is_wh = jnp.logical_or(c == 2, c == 3)
    stride = stride_ref[0, 0]

    for a in range(_NA):
        slab = jax.lax.dot_general(v2d[_C * a:_C * (a + 1), :], eye_c, dn,
                                   precision=jax.lax.Precision.HIGHEST,
                                   preferred_element_type=jnp.float32)
        e = jnp.exp(jnp.where(is_wh, slab, -slab))
        sig = 1.0 / (1.0 + e)
        mul = jnp.where(c == 2, _AW[a], _AH[a])
        res = jnp.where(c < 2, (sig + mesh) * stride,
                        jnp.where(c < 4, e * mul, sig))
        o_ref[0, pl.Slice(a, hw, _NA), :] = res


def kernel(x, img_dim):
    nB, nCh, nH, nW = x.shape
    stride = (img_dim[1].astype(jnp.float32) / nH).reshape(1, 1)
    grid = (nB, nH // _HBLK)
    out = pl.pallas_call(
        _decode_kernel,
        grid=grid,
        in_specs=[
            pl.BlockSpec((1, 1), lambda b, j: (0, 0),
                         memory_space=pltpu.SMEM),
            pl.BlockSpec((1, nCh, _HBLK, nW), lambda b, j: (b, 0, j, 0)),
        ],
        out_specs=pl.BlockSpec((1, _HBLK * nW * _NA, _C),
                               lambda b, j: (b, j, 0)),
        out_shape=jax.ShapeDtypeStruct((nB, nH * nW * _NA, _C), x.dtype),
    )(stride, x)
    return out
